# baseline (speedup 1.0000x reference)
import functools

import jax
import jax.numpy as jnp
from jax.experimental import pallas as pl
from jax.experimental.pallas import tpu as pltpu


def _max_kernel(nbrs_ref, *refs):
    k = len(refs) - 1
    x_refs, out_ref = refs[:k], refs[k]
    acc = x_refs[0][...]
    for r in x_refs[1:]:
        acc = jnp.maximum(acc, r[...])
    out_ref[...] = acc


def _index_map(k, j, nbrs):
    return (nbrs[k, j], 0, 0)


def kernel(x, neighbours):
    b, c, n_in = x.shape
    k, n_out = neighbours.shape
    bc = b * c
    xt = x.reshape(bc, n_in).T.reshape(n_in, 1, bc)

    in_specs = [
        pl.BlockSpec((1, 1, bc), functools.partial(_index_map, kk)) for kk in range(k)
    ]
    out = pl.pallas_call(
        _max_kernel,
        grid_spec=pltpu.PrefetchScalarGridSpec(
            num_scalar_prefetch=1,
            grid=(n_out,),
            in_specs=in_specs,
            out_specs=pl.BlockSpec((1, 1, bc), lambda j, nbrs: (j, 0, 0)),
        ),
        out_shape=jax.ShapeDtypeStruct((n_out, 1, bc), x.dtype),
    )(neighbours, *([xt] * k))
    return out.reshape(n_out, bc).T.reshape(b, c, n_out)


# speedup vs baseline: 4.2137x; 4.2137x over previous
import functools

import jax
import jax.numpy as jnp
from jax import lax
from jax.experimental import pallas as pl
from jax.experimental.pallas import tpu as pltpu
from jax.experimental.pallas import tpu_sc as plsc

_B, _C, _NIN, _NOUT, _K = 8, 384, 16384, 4096, 9
_BC = _B * _C
_NSUB = 16
_ROWS_PER_W = _BC // _NSUB
_R = 4
_NBATCH = _ROWS_PER_W // _R
_L = 16
_JW = _NOUT // 2
_NCHUNK = _JW // _L


def _sc_body(x_hbm, nbrs_hbm, out_hbm, idx_v, rows_v, out_v):
    cid = lax.axis_index("c")
    sid = lax.axis_index("s")
    base_row = sid * _ROWS_PER_W
    jbase = cid * _JW
    pltpu.sync_copy(nbrs_hbm.at[:, pl.ds(jbase, _JW)], idx_v)

    def batch_body(b, carry):
        row0 = base_row + b * _R
        for r in range(_R):
            pltpu.sync_copy(
                x_hbm.at[row0 + r], rows_v.at[pl.ds(r * _NIN, _NIN)]
            )

        def chunk_body(jc, carry2):
            j0 = jc * _L
            ivecs = [idx_v[k, pl.ds(j0, _L)] for k in range(_K)]
            for r in range(_R):
                off = [iv + (r * _NIN) for iv in ivecs]
                g = plsc.load_gather(rows_v, [off[0]])
                for k in range(1, _K):
                    g = jnp.maximum(g, plsc.load_gather(rows_v, [off[k]]))
                out_v[r, pl.ds(j0, _L)] = g
            return carry2

        lax.fori_loop(0, _NCHUNK, chunk_body, 0)
        pltpu.sync_copy(out_v, out_hbm.at[pl.ds(row0, _R), pl.ds(jbase, _JW)])
        return carry

    lax.fori_loop(0, _NBATCH, batch_body, 0)


_sc_call = functools.partial(
    pl.kernel,
    out_type=jax.ShapeDtypeStruct((_BC, _NOUT), jnp.float32),
    mesh=plsc.VectorSubcoreMesh(core_axis_name="c", subcore_axis_name="s"),
    compiler_params=pltpu.CompilerParams(needs_layout_passes=False),
    scratch_types=[
        pltpu.VMEM((_K, _JW), jnp.int32),
        pltpu.VMEM((_R * _NIN,), jnp.float32),
        pltpu.VMEM((_R, _JW), jnp.float32),
    ],
)(_sc_body)


def kernel(x, neighbours):
    b, c, n_in = x.shape
    xf = x.reshape(b * c, n_in)
    out = _sc_call(xf, neighbours)
    return out.reshape(b, c, _NOUT)


# speedup vs baseline: 9.4812x; 2.2501x over previous
import functools

import jax
import jax.numpy as jnp
from jax import lax
from jax.experimental import pallas as pl
from jax.experimental.pallas import tpu as pltpu
from jax.experimental.pallas import tpu_sc as plsc

_B, _C, _NIN, _NOUT, _K = 8, 384, 16384, 4096, 9
_BC = _B * _C
_NSUB = 16
_ROWS_PER_W = _BC // _NSUB
_R = 2
_NBATCH = _ROWS_PER_W // _R
_L = 16
_JW = _NOUT // 2
_NPAIR = _JW // (2 * _L)


def _sc_body(x_hbm, nbrs_hbm, out_hbm, idx_v, rows_a, rows_b, out_a, out_b,
             in_sem_a, in_sem_b, out_sem_a, out_sem_b):
    cid = lax.axis_index("c")
    sid = lax.axis_index("s")
    base_row = sid * _ROWS_PER_W
    jbase = cid * _JW
    pltpu.sync_copy(nbrs_hbm.at[:, pl.ds(cid * (_JW // 2), _JW // 2)], idx_v)

    def issue_in(bi, buf, sem):
        row0 = base_row + bi * _R
        for r in range(_R):
            pltpu.make_async_copy(
                x_hbm.at[row0 + r], buf.at[pl.ds(r * _NIN, _NIN)], sem
            ).start()

    def wait_in(buf, sem):
        pltpu.make_async_copy(x_hbm.at[0], buf.at[pl.ds(0, _NIN)], sem).wait()
        pltpu.make_async_copy(x_hbm.at[0], buf.at[pl.ds(0, _NIN)], sem).wait()

    def compute(buf, out_v):
        def pair_body(pc, carry):
            j0 = pc * (2 * _L)
            for k in range(_K):
                v = idx_v[k, pl.ds(pc * _L, _L)]
                lo = v & 0xFFFF
                hi = lax.shift_right_logical(v, 16)
                for r in range(_R):
                    glo = plsc.load_gather(buf, [lo + (r * _NIN)])
                    ghi = plsc.load_gather(buf, [hi + (r * _NIN)])
                    if k == 0:
                        acc[r][0], acc[r][1] = glo, ghi
                    else:
                        acc[r][0] = jnp.maximum(acc[r][0], glo)
                        acc[r][1] = jnp.maximum(acc[r][1], ghi)
            for r in range(_R):
                out_v[r, pl.ds(j0, _L)] = acc[r][0]
                out_v[r, pl.ds(j0 + _L, _L)] = acc[r][1]
            return carry

        acc = [[None, None] for _ in range(_R)]
        lax.fori_loop(0, _NPAIR, pair_body, 0)

    def issue_out(bi, out_v, sem):
        row0 = base_row + bi * _R
        pltpu.make_async_copy(
            out_v, out_hbm.at[pl.ds(row0, _R), pl.ds(jbase, _JW)], sem
        ).start()

    def wait_out(out_v, sem):
        pltpu.make_async_copy(
            out_v, out_hbm.at[pl.ds(0, _R), pl.ds(jbase, _JW)], sem
        ).wait()

    issue_in(0, rows_a, in_sem_a)
    issue_in(1, rows_b, in_sem_b)

    def pair_of_batches(p, carry):
        bi = 2 * p
        wait_in(rows_a, in_sem_a)

        @pl.when(bi + 2 < _NBATCH)
        def _():
            issue_in(bi + 2, rows_a, in_sem_a)

        @pl.when(p > 0)
        def _():
            wait_out(out_a, out_sem_a)

        compute(rows_a, out_a)
        issue_out(bi, out_a, out_sem_a)
        wait_in(rows_b, in_sem_b)

        @pl.when(bi + 3 < _NBATCH)
        def _():
            issue_in(bi + 3, rows_b, in_sem_b)

        @pl.when(p > 0)
        def _():
            wait_out(out_b, out_sem_b)

        compute(rows_b, out_b)
        issue_out(bi + 1, out_b, out_sem_b)
        return carry

    lax.fori_loop(0, _NBATCH // 2, pair_of_batches, 0)
    wait_out(out_a, out_sem_a)
    wait_out(out_b, out_sem_b)


_sc_call = functools.partial(
    pl.kernel,
    out_type=jax.ShapeDtypeStruct((_BC, _NOUT), jnp.float32),
    mesh=plsc.VectorSubcoreMesh(core_axis_name="c", subcore_axis_name="s"),
    compiler_params=pltpu.CompilerParams(needs_layout_passes=False),
    scratch_types=[
        pltpu.VMEM((_K, _JW // 2), jnp.int32),
        pltpu.VMEM((_R * _NIN,), jnp.float32),
        pltpu.VMEM((_R * _NIN,), jnp.float32),
        pltpu.VMEM((_R, _JW), jnp.float32),
        pltpu.VMEM((_R, _JW), jnp.float32),
        pltpu.SemaphoreType.DMA,
        pltpu.SemaphoreType.DMA,
        pltpu.SemaphoreType.DMA,
        pltpu.SemaphoreType.DMA,
    ],
)(_sc_body)


def kernel(x, neighbours):
    b, c, n_in = x.shape
    xf = x.reshape(b * c, n_in)
    nb = neighbours.reshape(_K, _NOUT // (2 * _L), 2, _L)
    packed = (nb[:, :, 0, :] | (nb[:, :, 1, :] << 16)).reshape(_K, _NOUT // 2)
    out = _sc_call(xf, packed)
    return out.reshape(b, c, _NOUT)
